# baseline (speedup 1.0000x reference)
import functools

import jax
import jax.numpy as jnp
from jax import lax
from jax.experimental import pallas as pl
from jax.experimental.pallas import tpu as pltpu
from jax.experimental.pallas import tpu_sc as plsc

_I = 16384
_J = 200
_D = 64
_V = 1000000
_NW = 32
_IB = _I // 128
_U = _J * _IB
_UPW = _U // _NW
_NBUF = 4
_G0 = 5
_GROUPS = (_UPW - 4 - _G0 + 1) // 4


def _make_gather():
    mesh = plsc.VectorSubcoreMesh(core_axis_name="c", subcore_axis_name="s")

    @functools.partial(
        pl.kernel,
        mesh=mesh,
        out_type=jax.ShapeDtypeStruct((_J, _D // 8, _IB, 1024), jnp.float32),
        scratch_types=[
            pltpu.VMEM((_NBUF, 128), jnp.int32),
            pltpu.VMEM((_NBUF, 128, _D), jnp.float32),
            pltpu.VMEM((_NBUF, _D // 8, 1024), jnp.float32),
            pltpu.SemaphoreType.DMA((_NBUF,)),
            pltpu.SemaphoreType.DMA((_NBUF,)),
            pltpu.SemaphoreType.DMA((_NBUF,)),
        ],
        compiler_params=pltpu.CompilerParams(use_tc_tiling_on_sc=False,
                                             needs_layout_passes=False),
    )
    def k(idx_hbm, table_hbm, out_hbm, idx_v, rows_v, perm_v,
          sem_i, sem_g, sem_o):
        wid = lax.axis_index("s") * 2 + lax.axis_index("c")
        base = wid * _UPW
        iotas = [lax.iota(jnp.int32, 16) + g * 16 for g in range(8)]

        def unit_ji(u):
            j = lax.shift_right_logical(u, 7)
            ib = lax.bitwise_and(u, 127)
            return j, ib

        def start_idx(u, b):
            j, ib = unit_ji(u)
            jb = lax.shift_right_logical(j, 3)
            jr = lax.bitwise_and(j, 7)
            pltpu.async_copy(idx_hbm.at[jb, ib, jr], idx_v.at[b],
                             sem_i.at[b])

        def wait_idx(b):
            pltpu.make_async_copy(idx_hbm.at[0, 0, 0], idx_v.at[b],
                                  sem_i.at[b]).wait()

        def start_gather(b):
            pltpu.async_copy(table_hbm.at[idx_v.at[b]], rows_v.at[b],
                             sem_g.at[b])

        def wait_gather(b):
            pltpu.make_async_copy(table_hbm.at[idx_v.at[b]], rows_v.at[b],
                                  sem_g.at[b]).wait()

        def permute(b):
            rows = rows_v.at[b]

            def body(t, carry):
                for dd in range(2):
                    d = t * 2 + dd
                    db = lax.shift_right_logical(d, 3)
                    dr = lax.bitwise_and(d, 7)
                    col = jnp.full((16,), d, jnp.int32)
                    for g in range(8):
                        vals = plsc.load_gather(rows, [iotas[g], col])
                        perm_v[b, db, pl.ds(dr * 128 + g * 16, 16)] = vals
                return carry

            lax.fori_loop(0, _D // 2, body, 0)

        def start_out(u, b):
            j, ib = unit_ji(u)
            pltpu.async_copy(perm_v.at[b], out_hbm.at[j, :, ib],
                             sem_o.at[b])

        def wait_out(b):
            pltpu.make_async_copy(perm_v.at[b], out_hbm.at[0, :, 0],
                                  sem_o.at[b]).wait()

        def finish(u, b, skip_out_wait=False):
            wait_gather(b)
            if not skip_out_wait:
                wait_out(b)
            permute(b)
            start_out(u, b)

        for b in range(_NBUF):
            start_idx(base + b, b)
        wait_idx(0)
        start_gather(0)

        def iteration(i, bi, bp, skip_out_wait):
            wait_idx(bi)
            start_gather(bi)
            finish(base + i - 1, bp, skip_out_wait)

            @pl.when(base + i + _NBUF - 1 < base + _UPW)
            def _():
                start_idx(base + i + _NBUF - 1, bp)

        for i in range(1, _G0):
            iteration(i, i % _NBUF, (i - 1) % _NBUF, skip_out_wait=True)

        def body(g, carry):
            for kk in range(4):
                iteration(_G0 + g * 4 + kk, (_G0 + kk) % _NBUF,
                          (_G0 + kk - 1) % _NBUF, skip_out_wait=False)
            return carry

        lax.fori_loop(0, _GROUPS, body, 0)

        for i in range(_G0 + _GROUPS * 4, _UPW):
            iteration(i, i % _NBUF, (i - 1) % _NBUF, skip_out_wait=False)
        finish(base + _UPW - 1, (_UPW - 1) % _NBUF)
        for b in range(_NBUF):
            wait_out(b)

    return k


_gather = _make_gather()


@jax.jit
def kernel(input_ids, weight):
    ids = input_ids if input_ids.dtype == jnp.int32 else input_ids.astype(jnp.int32)
    ids4 = ids.reshape(_I // 128, 128, _J // 8, 8).transpose(2, 0, 3, 1)
    out4 = _gather(ids4, weight)
    out = (out4.reshape(_J, _D // 8, _IB, 8, 128)
           .transpose(2, 4, 0, 1, 3)
           .reshape(_I, _J, _D))
    return out


# speedup vs baseline: 1.6698x; 1.6698x over previous
import functools

import jax
import jax.numpy as jnp
from jax import lax
from jax.experimental import pallas as pl
from jax.experimental.pallas import tpu as pltpu
from jax.experimental.pallas import tpu_sc as plsc

_I = 16384
_J = 200
_D = 64
_V = 1000000
_NW = 32
_IB = _I // 128
_U = _J * _IB
_UPW = _U // _NW
_NBUF = 4
_G0 = 5
_GROUPS = (_UPW - 4 - _G0 + 1) // 4


def _make_gather():
    mesh = plsc.VectorSubcoreMesh(core_axis_name="c", subcore_axis_name="s")

    @functools.partial(
        pl.kernel,
        mesh=mesh,
        out_type=jax.ShapeDtypeStruct((_J, _D // 8, _IB, 1024), jnp.float32),
        scratch_types=[
            pltpu.VMEM((_NBUF, 128), jnp.int32),
            pltpu.VMEM((_NBUF, 128, _D), jnp.float32),
            pltpu.VMEM((_NBUF, _D // 8, 1024), jnp.float32),
            pltpu.SemaphoreType.DMA((_NBUF,)),
            pltpu.SemaphoreType.DMA((_NBUF,)),
            pltpu.SemaphoreType.DMA((_NBUF,)),
        ],
        compiler_params=pltpu.CompilerParams(use_tc_tiling_on_sc=False,
                                             needs_layout_passes=False),
    )
    def k(idx_hbm, table_hbm, out_hbm, idx_v, rows_v, perm_v,
          sem_i, sem_g, sem_o):
        wid = lax.axis_index("s") * 2 + lax.axis_index("c")
        base = wid * _UPW
        iotas = [lax.iota(jnp.int32, 16) + g * 16 for g in range(8)]

        def unit_ji(u):
            j = lax.shift_right_logical(u, 7)
            ib = lax.bitwise_and(u, 127)
            return j, ib

        def start_idx(u, b):
            j, ib = unit_ji(u)
            jb = lax.shift_right_logical(j, 3)
            jr = lax.bitwise_and(j, 7)
            pltpu.async_copy(idx_hbm.at[jb, ib, jr], idx_v.at[b],
                             sem_i.at[b])

        def wait_idx(b):
            pltpu.make_async_copy(idx_hbm.at[0, 0, 0], idx_v.at[b],
                                  sem_i.at[b]).wait()

        def start_gather(b):
            pltpu.async_copy(table_hbm.at[idx_v.at[b]], rows_v.at[b],
                             sem_g.at[b])

        def wait_gather(b):
            pltpu.make_async_copy(table_hbm.at[idx_v.at[b]], rows_v.at[b],
                                  sem_g.at[b]).wait()

        def permute(b):
            rows = rows_v.at[b]

            @plsc.parallel_loop(0, _D, step=2, unroll=4)
            def body(t):
                for dd in range(2):
                    d = t + dd
                    db = lax.shift_right_logical(d, 3)
                    dr = lax.bitwise_and(d, 7)
                    col = jnp.full((16,), d, jnp.int32)
                    for g in range(8):
                        vals = plsc.load_gather(rows, [iotas[g], col])
                        perm_v[b, db, pl.ds(dr * 128 + g * 16, 16)] = vals

        def start_out(u, b):
            j, ib = unit_ji(u)
            pltpu.async_copy(perm_v.at[b], out_hbm.at[j, :, ib],
                             sem_o.at[b])

        def wait_out(b):
            pltpu.make_async_copy(perm_v.at[b], out_hbm.at[0, :, 0],
                                  sem_o.at[b]).wait()

        def finish(u, b, skip_out_wait=False):
            wait_gather(b)
            if not skip_out_wait:
                wait_out(b)
            permute(b)
            start_out(u, b)

        for b in range(_NBUF):
            start_idx(base + b, b)
        wait_idx(0)
        start_gather(0)

        def iteration(i, bi, bp, skip_out_wait):
            wait_idx(bi)
            start_gather(bi)
            finish(base + i - 1, bp, skip_out_wait)

            @pl.when(base + i + _NBUF - 1 < base + _UPW)
            def _():
                start_idx(base + i + _NBUF - 1, bp)

        for i in range(1, _G0):
            iteration(i, i % _NBUF, (i - 1) % _NBUF, skip_out_wait=True)

        def body(g, carry):
            for kk in range(4):
                iteration(_G0 + g * 4 + kk, (_G0 + kk) % _NBUF,
                          (_G0 + kk - 1) % _NBUF, skip_out_wait=False)
            return carry

        lax.fori_loop(0, _GROUPS, body, 0)

        for i in range(_G0 + _GROUPS * 4, _UPW):
            iteration(i, i % _NBUF, (i - 1) % _NBUF, skip_out_wait=False)
        finish(base + _UPW - 1, (_UPW - 1) % _NBUF)
        for b in range(_NBUF):
            wait_out(b)

    return k


_gather = _make_gather()


@jax.jit
def kernel(input_ids, weight):
    ids = input_ids if input_ids.dtype == jnp.int32 else input_ids.astype(jnp.int32)
    ids4 = ids.reshape(_I // 128, 128, _J // 8, 8).transpose(2, 0, 3, 1)
    out4 = _gather(ids4, weight)
    out = (out4.reshape(_J, _D // 8, _IB, 8, 128)
           .transpose(2, 4, 0, 1, 3)
           .reshape(_I, _J, _D))
    return out


# speedup vs baseline: 3.3676x; 2.0168x over previous
import functools

import jax
import jax.numpy as jnp
from jax import lax
from jax.experimental import pallas as pl
from jax.experimental.pallas import tpu as pltpu
from jax.experimental.pallas import tpu_sc as plsc

_I = 16384
_J = 200
_D = 64
_V = 1000000
_NW = 32
_IB = _I // 128
_U = _J * _IB
_UPW = _U // _NW
_NBUF = 4
_G0 = 5
_GROUPS = (_UPW - 4 - _G0 + 1) // 4


def _make_gather():
    mesh = plsc.VectorSubcoreMesh(core_axis_name="c", subcore_axis_name="s")

    @functools.partial(
        pl.kernel,
        mesh=mesh,
        out_type=jax.ShapeDtypeStruct((_J, _D // 8, _IB, 1024), jnp.float32),
        scratch_types=[
            pltpu.VMEM((_NBUF, 128), jnp.int32),
            pltpu.VMEM((_NBUF, 128, _D), jnp.float32),
            pltpu.VMEM((_NBUF, _D // 8, 1024), jnp.float32),
            pltpu.SemaphoreType.DMA((_NBUF,)),
            pltpu.SemaphoreType.DMA((_NBUF,)),
            pltpu.SemaphoreType.DMA((_NBUF,)),
        ],
        compiler_params=pltpu.CompilerParams(use_tc_tiling_on_sc=False,
                                             needs_layout_passes=False),
    )
    def k(idx_hbm, table_hbm, out_hbm, idx_v, rows_v, perm_v,
          sem_i, sem_g, sem_o):
        wid = lax.axis_index("s") * 2 + lax.axis_index("c")
        base = wid * _UPW
        iota16 = lax.iota(jnp.int32, 16)
        iotas = [iota16 + g * 16 for g in range(8)]

        def unit_ji(u):
            j = lax.shift_right_logical(u, 7)
            ib = lax.bitwise_and(u, 127)
            return j, ib

        def start_idx(u, b):
            j, ib = unit_ji(u)
            jb = lax.shift_right_logical(j, 3)
            jr = lax.bitwise_and(j, 7)
            pltpu.async_copy(idx_hbm.at[jb, ib, jr], idx_v.at[b],
                             sem_i.at[b])

        def wait_idx(b):
            pltpu.make_async_copy(idx_hbm.at[0, 0, 0], idx_v.at[b],
                                  sem_i.at[b]).wait()

        def start_gather(b):
            pltpu.async_copy(table_hbm.at[idx_v.at[b]], rows_v.at[b],
                             sem_g.at[b])

        def wait_gather(b):
            pltpu.make_async_copy(table_hbm.at[idx_v.at[b]], rows_v.at[b],
                                  sem_g.at[b]).wait()

        def permute(b):
            rows = rows_v.at[b]
            perm = perm_v.at[b]

            @plsc.parallel_loop(0, 16, step=1, unroll=2)
            def sbody(s):
                rot = lax.bitwise_and(iota16 + s, 15)
                rot_hi = lax.shift_right_logical(rot, 3)
                rot_lo = lax.shift_left(lax.bitwise_and(rot, 7), 7)
                cols = [rot + d0 for d0 in range(0, _D, 16)]
                dbs = [rot_hi + (d0 >> 3) for d0 in range(0, _D, 16)]
                for g in range(8):
                    lo_g = rot_lo + iotas[g]
                    for t in range(_D // 16):
                        vals = plsc.load_gather(rows, [iotas[g], cols[t]])
                        plsc.store_scatter(perm, [dbs[t], lo_g], vals)

        def start_out(u, b):
            j, ib = unit_ji(u)
            pltpu.async_copy(perm_v.at[b], out_hbm.at[j, :, ib],
                             sem_o.at[b])

        def wait_out(b):
            pltpu.make_async_copy(perm_v.at[b], out_hbm.at[0, :, 0],
                                  sem_o.at[b]).wait()

        def finish(u, b, skip_out_wait=False):
            wait_gather(b)
            if not skip_out_wait:
                wait_out(b)
            permute(b)
            start_out(u, b)

        for b in range(_NBUF):
            start_idx(base + b, b)
        wait_idx(0)
        start_gather(0)

        def iteration(i, bi, bp, skip_out_wait):
            wait_idx(bi)
            start_gather(bi)
            finish(base + i - 1, bp, skip_out_wait)

            @pl.when(base + i + _NBUF - 1 < base + _UPW)
            def _():
                start_idx(base + i + _NBUF - 1, bp)

        for i in range(1, _G0):
            iteration(i, i % _NBUF, (i - 1) % _NBUF, skip_out_wait=True)

        def body(g, carry):
            for kk in range(4):
                iteration(_G0 + g * 4 + kk, (_G0 + kk) % _NBUF,
                          (_G0 + kk - 1) % _NBUF, skip_out_wait=False)
            return carry

        lax.fori_loop(0, _GROUPS, body, 0)

        for i in range(_G0 + _GROUPS * 4, _UPW):
            iteration(i, i % _NBUF, (i - 1) % _NBUF, skip_out_wait=False)
        finish(base + _UPW - 1, (_UPW - 1) % _NBUF)
        for b in range(_NBUF):
            wait_out(b)

    return k


_gather = _make_gather()


@jax.jit
def kernel(input_ids, weight):
    ids = input_ids if input_ids.dtype == jnp.int32 else input_ids.astype(jnp.int32)
    ids4 = ids.reshape(_I // 128, 128, _J // 8, 8).transpose(2, 0, 3, 1)
    out4 = _gather(ids4, weight)
    out = (out4.reshape(_J, _D // 8, _IB, 8, 128)
           .transpose(2, 4, 0, 1, 3)
           .reshape(_I, _J, _D))
    return out
